# baseline (speedup 1.0000x reference)
import functools
import jax
import jax.numpy as jnp
from jax import lax
from jax.experimental import pallas as pl
from jax.experimental.pallas import tpu as pltpu
from jax.experimental.pallas import tpu_sc as plsc

N_NODES = 10000
N_EDGES = 320000
D_FEAT = 128
N_CLS = 10

NC = 2
NS = 16
NW = NC * NS

K = 80
CHUNKS = N_EDGES // (NW * K)
RPT = N_NODES // NS
DK = K
DCHUNKS = CHUNKS

_sc_mesh = plsc.VectorSubcoreMesh(core_axis_name="c", subcore_axis_name="s")


@functools.partial(
    pl.kernel,
    out_type=jax.ShapeDtypeStruct((NC, N_NODES, 16), jnp.float32),
    mesh=_sc_mesh,
    compiler_params=pltpu.CompilerParams(use_tc_tiling_on_sc=False),
    scratch_types=[
        pltpu.VMEM((DCHUNKS, DK), jnp.int32),
        pltpu.VMEM((DK, 16), jnp.float32),
        pltpu.VMEM_SHARED((N_NODES, 16), jnp.float32),
        pltpu.SemaphoreType.DMA,
    ],
)
def _deg_pass(edges_hbm, ones_hbm, zeros_hbm, out_hbm, dst_i, ones_v, deg_sh,
              dsem):
    cid = lax.axis_index("c")
    sid = lax.axis_index("s")
    r0 = sid * RPT
    wid = sid * NC + cid

    pltpu.sync_copy(edges_hbm.at[1, wid], dst_i)

    @pl.when(cid == 0)
    def _():
        pltpu.sync_copy(ones_hbm, deg_sh.at[pl.ds(r0, RPT)])

    @pl.when(cid != 0)
    def _():
        pltpu.sync_copy(zeros_hbm, deg_sh.at[pl.ds(r0, RPT)])

    pltpu.sync_copy(ones_hbm.at[pl.ds(0, DK)], ones_v)
    plsc.subcore_barrier()

    def body(j, carry):
        pltpu.async_copy(ones_v, deg_sh.at[dst_i.at[j]], dsem, add=True)
        return carry

    lax.fori_loop(0, DCHUNKS, body, 0)

    def drain(j, carry):
        pltpu.make_async_copy(ones_hbm.at[pl.ds(0, DK)], ones_v, dsem).wait()
        return carry

    lax.fori_loop(0, DCHUNKS, drain, 0)
    plsc.subcore_barrier()
    pltpu.sync_copy(deg_sh.at[pl.ds(r0, RPT)], out_hbm.at[cid, pl.ds(r0, RPT)])


@functools.partial(
    pl.kernel,
    out_type=jax.ShapeDtypeStruct((NC, N_NODES, D_FEAT), jnp.float32),
    mesh=_sc_mesh,
    compiler_params=pltpu.CompilerParams(use_tc_tiling_on_sc=False),
    scratch_types=[
        pltpu.VMEM((CHUNKS, K), jnp.int32),
        pltpu.VMEM((CHUNKS, K), jnp.int32),
        pltpu.VMEM((K, D_FEAT), jnp.float32),
        pltpu.VMEM((K, D_FEAT), jnp.float32),
        pltpu.VMEM((K, D_FEAT), jnp.float32),
        pltpu.VMEM_SHARED((N_NODES, D_FEAT), jnp.float32),
        pltpu.SemaphoreType.DMA,
        pltpu.SemaphoreType.DMA,
        pltpu.SemaphoreType.DMA,
    ],
)
def _edge_pass(y_hbm, edges_hbm, zeros_hbm, out_hbm,
               src_i, dst_i, rows0, rows1, rows2, agg_sh, gsem0, gsem1, gsem2):
    cid = lax.axis_index("c")
    sid = lax.axis_index("s")
    r0 = sid * RPT
    wid = sid * NC + cid

    pltpu.sync_copy(edges_hbm.at[0, wid], src_i)
    pltpu.sync_copy(edges_hbm.at[1, wid], dst_i)

    @pl.when(cid == 0)
    def _():
        pltpu.sync_copy(y_hbm.at[pl.ds(r0, RPT)], agg_sh.at[pl.ds(r0, RPT)])

    @pl.when(cid != 0)
    def _():
        pltpu.sync_copy(zeros_hbm, agg_sh.at[pl.ds(r0, RPT)])

    plsc.subcore_barrier()

    bufs = (rows0, rows1, rows2)
    gsems = (gsem0, gsem1, gsem2)

    def fire(c, b):
        pltpu.async_copy(y_hbm.at[src_i.at[c]], bufs[b], gsems[b])

    def wait_scatter(c, b):
        pltpu.make_async_copy(y_hbm.at[pl.ds(0, K)], bufs[b], gsems[b]).wait()
        pltpu.sync_copy(bufs[b], agg_sh.at[dst_i.at[c]], add=True)

    fire(0, 0)
    fire(1, 1)
    fire(2, 2)

    def body(g, carry):
        c0 = 3 * g
        wait_scatter(c0, 0)
        fire(c0 + 3, 0)
        wait_scatter(c0 + 1, 1)
        fire(c0 + 4, 1)
        wait_scatter(c0 + 2, 2)
        fire(c0 + 5, 2)
        return carry

    lax.fori_loop(0, (CHUNKS - 5) // 3, body, 0)
    wait_scatter(CHUNKS - 5, 0)
    fire(CHUNKS - 2, 0)
    wait_scatter(CHUNKS - 4, 1)
    fire(CHUNKS - 1, 1)
    wait_scatter(CHUNKS - 3, 2)
    wait_scatter(CHUNKS - 2, 0)
    wait_scatter(CHUNKS - 1, 1)
    plsc.subcore_barrier()
    pltpu.sync_copy(agg_sh.at[pl.ds(r0, RPT)], out_hbm.at[cid, pl.ds(r0, RPT)])


_R = 400


def _mm1_body(x_ref, w_ref, deg_ref, y_ref, dinv_ref):
    d = deg_ref[0] + deg_ref[1]
    dinv = lax.rsqrt(d)
    dinv_ref[...] = dinv
    xw = jnp.dot(x_ref[...], w_ref[...], preferred_element_type=jnp.float32)
    y_ref[...] = xw * dinv[:, 0:1]


def _mm2_body(agg_ref, dinv_ref, b_ref, w_ref, y_ref):
    dinv = dinv_ref[...][:, 0:1]
    h = jnp.maximum((agg_ref[0] + agg_ref[1]) * dinv + b_ref[...], 0.0)
    y_ref[...] = jnp.dot(h, w_ref[...], preferred_element_type=jnp.float32) * dinv


def _mm3_body(agg_ref, dinv_ref, b_ref, wl_ref, bl_ref, out_ref):
    dinv = dinv_ref[...][:, 0:1]
    h = jnp.maximum((agg_ref[0] + agg_ref[1]) * dinv + b_ref[...], 0.0)
    logits = jnp.dot(h, wl_ref[...], preferred_element_type=jnp.float32) + bl_ref[...]
    col = lax.broadcasted_iota(jnp.int32, logits.shape, 1)
    valid = col < N_CLS
    masked = jnp.where(valid, logits, -jnp.inf)
    m = jnp.max(masked, axis=1, keepdims=True)
    e = jnp.where(valid, jnp.exp(logits - m), 0.0)
    lse = jnp.log(jnp.sum(e, axis=1, keepdims=True)) + m
    out_ref[...] = (logits - lse)[:, :N_CLS]


def kernel(x, edge_index, W1, b1, W2, b2, Wl, bl):
    edges = edge_index.reshape(2, NW, CHUNKS, K)

    ones16 = jnp.ones((RPT, 16), jnp.float32)
    zeros16 = jnp.zeros((RPT, 16), jnp.float32)
    zerosD = jnp.zeros((RPT, D_FEAT), jnp.float32)

    deg2 = _deg_pass(edges, ones16, zeros16)

    grid = (N_NODES // _R,)
    y1, dinv16 = pl.pallas_call(
        _mm1_body,
        grid=grid,
        in_specs=[
            pl.BlockSpec((_R, D_FEAT), lambda i: (i, 0)),
            pl.BlockSpec((D_FEAT, D_FEAT), lambda i: (0, 0)),
            pl.BlockSpec((NC, _R, 16), lambda i: (0, i, 0)),
        ],
        out_specs=[
            pl.BlockSpec((_R, D_FEAT), lambda i: (i, 0)),
            pl.BlockSpec((_R, 16), lambda i: (i, 0)),
        ],
        out_shape=[
            jax.ShapeDtypeStruct((N_NODES, D_FEAT), jnp.float32),
            jax.ShapeDtypeStruct((N_NODES, 16), jnp.float32),
        ],
    )(x, W1, deg2)

    agg1 = _edge_pass(y1, edges, zerosD)

    b1r = b1.reshape(1, D_FEAT)
    y2 = pl.pallas_call(
        _mm2_body,
        grid=grid,
        in_specs=[
            pl.BlockSpec((NC, _R, D_FEAT), lambda i: (0, i, 0)),
            pl.BlockSpec((_R, 16), lambda i: (i, 0)),
            pl.BlockSpec((1, D_FEAT), lambda i: (0, 0)),
            pl.BlockSpec((D_FEAT, D_FEAT), lambda i: (0, 0)),
        ],
        out_specs=pl.BlockSpec((_R, D_FEAT), lambda i: (i, 0)),
        out_shape=jax.ShapeDtypeStruct((N_NODES, D_FEAT), jnp.float32),
    )(agg1, dinv16, b1r, W2)

    agg2 = _edge_pass(y2, edges, zerosD)

    b2r = b2.reshape(1, D_FEAT)
    Wlp = jnp.zeros((D_FEAT, D_FEAT), jnp.float32).at[:, :N_CLS].set(Wl)
    blp = jnp.zeros((1, D_FEAT), jnp.float32).at[0, :N_CLS].set(bl)
    outp = pl.pallas_call(
        _mm3_body,
        grid=grid,
        in_specs=[
            pl.BlockSpec((NC, _R, D_FEAT), lambda i: (0, i, 0)),
            pl.BlockSpec((_R, 16), lambda i: (i, 0)),
            pl.BlockSpec((1, D_FEAT), lambda i: (0, 0)),
            pl.BlockSpec((D_FEAT, D_FEAT), lambda i: (0, 0)),
            pl.BlockSpec((1, D_FEAT), lambda i: (0, 0)),
        ],
        out_specs=pl.BlockSpec((_R, N_CLS), lambda i: (i, 0)),
        out_shape=jax.ShapeDtypeStruct((N_NODES, N_CLS), jnp.float32),
    )(agg2, dinv16, b2r, Wlp, blp)

    return outp


# speedup vs baseline: 1.1190x; 1.1190x over previous
import functools
import jax
import jax.numpy as jnp
from jax import lax
from jax.experimental import pallas as pl
from jax.experimental.pallas import tpu as pltpu
from jax.experimental.pallas import tpu_sc as plsc

N_NODES = 10000
N_EDGES = 320000
D_FEAT = 128
N_CLS = 10

NC = 2
NS = 16
NW = NC * NS

K = 80
CHUNKS = N_EDGES // (NW * K)
RPT = N_NODES // NS
DK = K
DCHUNKS = CHUNKS

_sc_mesh = plsc.VectorSubcoreMesh(core_axis_name="c", subcore_axis_name="s")


@functools.partial(
    pl.kernel,
    out_type=jax.ShapeDtypeStruct((NC, N_NODES, 16), jnp.float32),
    mesh=_sc_mesh,
    compiler_params=pltpu.CompilerParams(use_tc_tiling_on_sc=False),
    scratch_types=[
        pltpu.VMEM((DCHUNKS, DK), jnp.int32),
        pltpu.VMEM((DK, 16), jnp.float32),
        pltpu.VMEM_SHARED((N_NODES, 16), jnp.float32),
        pltpu.SemaphoreType.DMA,
    ],
)
def _deg_pass(edges_hbm, ones_hbm, zeros_hbm, out_hbm, dst_i, ones_v, deg_sh,
              dsem):
    cid = lax.axis_index("c")
    sid = lax.axis_index("s")
    r0 = sid * RPT
    wid = sid * NC + cid

    pltpu.sync_copy(edges_hbm.at[1, wid], dst_i)

    @pl.when(cid == 0)
    def _():
        pltpu.sync_copy(ones_hbm, deg_sh.at[pl.ds(r0, RPT)])

    @pl.when(cid != 0)
    def _():
        pltpu.sync_copy(zeros_hbm, deg_sh.at[pl.ds(r0, RPT)])

    pltpu.sync_copy(ones_hbm.at[pl.ds(0, DK)], ones_v)
    plsc.subcore_barrier()

    def body(j, carry):
        pltpu.async_copy(ones_v, deg_sh.at[dst_i.at[j]], dsem, add=True)
        return carry

    lax.fori_loop(0, DCHUNKS, body, 0)

    def drain(j, carry):
        pltpu.make_async_copy(ones_hbm.at[pl.ds(0, DK)], ones_v, dsem).wait()
        return carry

    lax.fori_loop(0, DCHUNKS, drain, 0)
    plsc.subcore_barrier()
    pltpu.sync_copy(deg_sh.at[pl.ds(r0, RPT)], out_hbm.at[cid, pl.ds(r0, RPT)])


@functools.partial(
    pl.kernel,
    out_type=jax.ShapeDtypeStruct((NC, N_NODES, D_FEAT), jnp.float32),
    mesh=_sc_mesh,
    compiler_params=pltpu.CompilerParams(use_tc_tiling_on_sc=False),
    scratch_types=[
        pltpu.VMEM((CHUNKS, K), jnp.int32),
        pltpu.VMEM((CHUNKS, K), jnp.int32),
        pltpu.VMEM((K, D_FEAT), jnp.float32),
        pltpu.VMEM((K, D_FEAT), jnp.float32),
        pltpu.VMEM((K, D_FEAT), jnp.float32),
        pltpu.VMEM_SHARED((N_NODES, D_FEAT), jnp.float32),
        pltpu.SemaphoreType.DMA,
        pltpu.SemaphoreType.DMA,
        pltpu.SemaphoreType.DMA,
    ],
)
def _edge_pass(y_hbm, edges_hbm, zeros_hbm, out_hbm,
               src_i, dst_i, rows0, rows1, rows2, agg_sh, gsem0, gsem1, gsem2):
    cid = lax.axis_index("c")
    sid = lax.axis_index("s")
    r0 = sid * RPT
    wid = sid * NC + cid

    pltpu.sync_copy(edges_hbm.at[0, wid], src_i)
    pltpu.sync_copy(edges_hbm.at[1, wid], dst_i)

    @pl.when(cid == 0)
    def _():
        pltpu.sync_copy(y_hbm.at[pl.ds(r0, RPT)], agg_sh.at[pl.ds(r0, RPT)])

    @pl.when(cid != 0)
    def _():
        pltpu.sync_copy(zeros_hbm, agg_sh.at[pl.ds(r0, RPT)])

    plsc.subcore_barrier()

    bufs = (rows0, rows1, rows2)
    gsems = (gsem0, gsem1, gsem2)

    def fire(c, b):
        pltpu.async_copy(y_hbm.at[src_i.at[c]], bufs[b], gsems[b])

    def wait_scatter(c, b):
        pltpu.make_async_copy(y_hbm.at[pl.ds(0, K)], bufs[b], gsems[b]).wait()
        pltpu.sync_copy(bufs[b], agg_sh.at[dst_i.at[c]], add=True)

    fire(0, 0)
    fire(1, 1)
    fire(2, 2)

    def body(g, carry):
        c0 = 3 * g
        wait_scatter(c0, 0)
        fire(c0 + 3, 0)
        wait_scatter(c0 + 1, 1)
        fire(c0 + 4, 1)
        wait_scatter(c0 + 2, 2)
        fire(c0 + 5, 2)
        return carry

    lax.fori_loop(0, (CHUNKS - 5) // 3, body, 0)
    wait_scatter(CHUNKS - 5, 0)
    fire(CHUNKS - 2, 0)
    wait_scatter(CHUNKS - 4, 1)
    fire(CHUNKS - 1, 1)
    wait_scatter(CHUNKS - 3, 2)
    wait_scatter(CHUNKS - 2, 0)
    wait_scatter(CHUNKS - 1, 1)
    plsc.subcore_barrier()
    pltpu.sync_copy(agg_sh.at[pl.ds(r0, RPT)], out_hbm.at[cid, pl.ds(r0, RPT)])


_R = 2000


def _mm1_body(x_ref, w_ref, deg_ref, y_ref, dinv_ref):
    d = deg_ref[0] + deg_ref[1]
    dinv = lax.rsqrt(d)
    dinv_ref[...] = dinv
    xw = jnp.dot(x_ref[...], w_ref[...], preferred_element_type=jnp.float32)
    y_ref[...] = xw * dinv[:, 0:1]


def _mm2_body(agg_ref, dinv_ref, b_ref, w_ref, y_ref):
    dinv = dinv_ref[...][:, 0:1]
    h = jnp.maximum((agg_ref[0] + agg_ref[1]) * dinv + b_ref[...], 0.0)
    y_ref[...] = jnp.dot(h, w_ref[...], preferred_element_type=jnp.float32) * dinv


def _mm3_body(agg_ref, dinv_ref, b_ref, wl_ref, bl_ref, out_ref):
    dinv = dinv_ref[...][:, 0:1]
    h = jnp.maximum((agg_ref[0] + agg_ref[1]) * dinv + b_ref[...], 0.0)
    logits = jnp.dot(h, wl_ref[...], preferred_element_type=jnp.float32) + bl_ref[...]
    col = lax.broadcasted_iota(jnp.int32, logits.shape, 1)
    valid = col < N_CLS
    masked = jnp.where(valid, logits, -jnp.inf)
    m = jnp.max(masked, axis=1, keepdims=True)
    e = jnp.where(valid, jnp.exp(logits - m), 0.0)
    lse = jnp.log(jnp.sum(e, axis=1, keepdims=True)) + m
    out_ref[...] = (logits - lse)[:, :N_CLS]


def kernel(x, edge_index, W1, b1, W2, b2, Wl, bl):
    edges = edge_index.reshape(2, NW, CHUNKS, K)

    ones16 = jnp.ones((RPT, 16), jnp.float32)
    zeros16 = jnp.zeros((RPT, 16), jnp.float32)
    zerosD = jnp.zeros((RPT, D_FEAT), jnp.float32)

    deg2 = _deg_pass(edges, ones16, zeros16)

    grid = (N_NODES // _R,)
    y1, dinv16 = pl.pallas_call(
        _mm1_body,
        grid=grid,
        in_specs=[
            pl.BlockSpec((_R, D_FEAT), lambda i: (i, 0)),
            pl.BlockSpec((D_FEAT, D_FEAT), lambda i: (0, 0)),
            pl.BlockSpec((NC, _R, 16), lambda i: (0, i, 0)),
        ],
        out_specs=[
            pl.BlockSpec((_R, D_FEAT), lambda i: (i, 0)),
            pl.BlockSpec((_R, 16), lambda i: (i, 0)),
        ],
        out_shape=[
            jax.ShapeDtypeStruct((N_NODES, D_FEAT), jnp.float32),
            jax.ShapeDtypeStruct((N_NODES, 16), jnp.float32),
        ],
    )(x, W1, deg2)

    agg1 = _edge_pass(y1, edges, zerosD)

    b1r = b1.reshape(1, D_FEAT)
    y2 = pl.pallas_call(
        _mm2_body,
        grid=grid,
        in_specs=[
            pl.BlockSpec((NC, _R, D_FEAT), lambda i: (0, i, 0)),
            pl.BlockSpec((_R, 16), lambda i: (i, 0)),
            pl.BlockSpec((1, D_FEAT), lambda i: (0, 0)),
            pl.BlockSpec((D_FEAT, D_FEAT), lambda i: (0, 0)),
        ],
        out_specs=pl.BlockSpec((_R, D_FEAT), lambda i: (i, 0)),
        out_shape=jax.ShapeDtypeStruct((N_NODES, D_FEAT), jnp.float32),
    )(agg1, dinv16, b1r, W2)

    agg2 = _edge_pass(y2, edges, zerosD)

    b2r = b2.reshape(1, D_FEAT)
    Wlp = jnp.zeros((D_FEAT, D_FEAT), jnp.float32).at[:, :N_CLS].set(Wl)
    blp = jnp.zeros((1, D_FEAT), jnp.float32).at[0, :N_CLS].set(bl)
    outp = pl.pallas_call(
        _mm3_body,
        grid=grid,
        in_specs=[
            pl.BlockSpec((NC, _R, D_FEAT), lambda i: (0, i, 0)),
            pl.BlockSpec((_R, 16), lambda i: (i, 0)),
            pl.BlockSpec((1, D_FEAT), lambda i: (0, 0)),
            pl.BlockSpec((D_FEAT, D_FEAT), lambda i: (0, 0)),
            pl.BlockSpec((1, D_FEAT), lambda i: (0, 0)),
        ],
        out_specs=pl.BlockSpec((_R, N_CLS), lambda i: (i, 0)),
        out_shape=jax.ShapeDtypeStruct((N_NODES, N_CLS), jnp.float32),
    )(agg2, dinv16, b2r, Wlp, blp)

    return outp


# speedup vs baseline: 1.1381x; 1.0171x over previous
import functools
import jax
import jax.numpy as jnp
from jax import lax
from jax.experimental import pallas as pl
from jax.experimental.pallas import tpu as pltpu
from jax.experimental.pallas import tpu_sc as plsc

N_NODES = 10000
N_EDGES = 320000
D_FEAT = 128
N_CLS = 10

NC = 2
NS = 16
NW = NC * NS

K = 80
CHUNKS = N_EDGES // (NW * K)
RPT = N_NODES // NS
DK = K
DCHUNKS = CHUNKS

_sc_mesh = plsc.VectorSubcoreMesh(core_axis_name="c", subcore_axis_name="s")


@functools.partial(
    pl.kernel,
    out_type=jax.ShapeDtypeStruct((NC, N_NODES, 16), jnp.float32),
    mesh=_sc_mesh,
    compiler_params=pltpu.CompilerParams(use_tc_tiling_on_sc=False),
    scratch_types=[
        pltpu.VMEM((DCHUNKS, DK), jnp.int32),
        pltpu.VMEM((DK, 16), jnp.float32),
        pltpu.VMEM_SHARED((N_NODES, 16), jnp.float32),
        pltpu.SemaphoreType.DMA,
    ],
)
def _deg_pass(edges_hbm, ones_hbm, zeros_hbm, out_hbm, dst_i, ones_v, deg_sh,
              dsem):
    cid = lax.axis_index("c")
    sid = lax.axis_index("s")
    r0 = sid * RPT
    wid = sid * NC + cid

    pltpu.sync_copy(edges_hbm.at[1, wid], dst_i)

    @pl.when(cid == 0)
    def _():
        pltpu.sync_copy(ones_hbm, deg_sh.at[pl.ds(r0, RPT)])

    @pl.when(cid != 0)
    def _():
        pltpu.sync_copy(zeros_hbm, deg_sh.at[pl.ds(r0, RPT)])

    pltpu.sync_copy(ones_hbm.at[pl.ds(0, DK)], ones_v)
    plsc.subcore_barrier()

    def body(j, carry):
        pltpu.async_copy(ones_v, deg_sh.at[dst_i.at[j]], dsem, add=True)
        return carry

    lax.fori_loop(0, DCHUNKS, body, 0)

    def drain(j, carry):
        pltpu.make_async_copy(ones_hbm.at[pl.ds(0, DK)], ones_v, dsem).wait()
        return carry

    lax.fori_loop(0, DCHUNKS, drain, 0)
    plsc.subcore_barrier()
    pltpu.sync_copy(deg_sh.at[pl.ds(r0, RPT)], out_hbm.at[cid, pl.ds(r0, RPT)])


@functools.partial(
    pl.kernel,
    out_type=jax.ShapeDtypeStruct((NC, N_NODES, D_FEAT), jnp.float32),
    mesh=_sc_mesh,
    compiler_params=pltpu.CompilerParams(use_tc_tiling_on_sc=False),
    scratch_types=[
        pltpu.VMEM((CHUNKS, K), jnp.int32),
        pltpu.VMEM((CHUNKS, K), jnp.int32),
        pltpu.VMEM((K, D_FEAT), jnp.float32),
        pltpu.VMEM((K, D_FEAT), jnp.float32),
        pltpu.VMEM((K, D_FEAT), jnp.float32),
        pltpu.VMEM_SHARED((N_NODES, D_FEAT), jnp.float32),
        pltpu.SemaphoreType.DMA,
        pltpu.SemaphoreType.DMA,
        pltpu.SemaphoreType.DMA,
    ],
)
def _edge_pass(y_hbm, edges_hbm, zeros_hbm, out_hbm,
               src_i, dst_i, rows0, rows1, rows2, agg_sh, gsem0, gsem1, gsem2):
    cid = lax.axis_index("c")
    sid = lax.axis_index("s")
    r0 = sid * RPT
    wid = sid * NC + cid

    pltpu.sync_copy(edges_hbm.at[0, wid], src_i)
    pltpu.sync_copy(edges_hbm.at[1, wid], dst_i)

    @pl.when(cid == 0)
    def _():
        pltpu.sync_copy(y_hbm.at[pl.ds(r0, RPT)], agg_sh.at[pl.ds(r0, RPT)])

    @pl.when(cid != 0)
    def _():
        pltpu.sync_copy(zeros_hbm, agg_sh.at[pl.ds(r0, RPT)])

    plsc.subcore_barrier()

    bufs = (rows0, rows1, rows2)
    gsems = (gsem0, gsem1, gsem2)

    def fire(c, b):
        pltpu.async_copy(y_hbm.at[src_i.at[c]], bufs[b], gsems[b])

    def wait_scatter(c, b):
        pltpu.make_async_copy(y_hbm.at[pl.ds(0, K)], bufs[b], gsems[b]).wait()
        pltpu.sync_copy(bufs[b], agg_sh.at[dst_i.at[c]], add=True)

    fire(0, 0)
    fire(1, 1)
    fire(2, 2)

    def body(g, carry):
        c0 = 3 * g
        wait_scatter(c0, 0)
        fire(c0 + 3, 0)
        wait_scatter(c0 + 1, 1)
        fire(c0 + 4, 1)
        wait_scatter(c0 + 2, 2)
        fire(c0 + 5, 2)
        return carry

    lax.fori_loop(0, (CHUNKS - 5) // 3, body, 0)
    wait_scatter(CHUNKS - 5, 0)
    fire(CHUNKS - 2, 0)
    wait_scatter(CHUNKS - 4, 1)
    fire(CHUNKS - 1, 1)
    wait_scatter(CHUNKS - 3, 2)
    wait_scatter(CHUNKS - 2, 0)
    wait_scatter(CHUNKS - 1, 1)
    plsc.subcore_barrier()
    pltpu.sync_copy(agg_sh.at[pl.ds(r0, RPT)], out_hbm.at[cid, pl.ds(r0, RPT)])


_R = 5000


def _mm1_body(x_ref, w_ref, deg_ref, y_ref, dinv_ref):
    d = deg_ref[0] + deg_ref[1]
    dinv = lax.rsqrt(d)
    dinv_ref[...] = dinv
    xw = jnp.dot(x_ref[...], w_ref[...], preferred_element_type=jnp.float32)
    y_ref[...] = xw * dinv[:, 0:1]


def _mm2_body(agg_ref, dinv_ref, b_ref, w_ref, y_ref):
    dinv = dinv_ref[...][:, 0:1]
    h = jnp.maximum((agg_ref[0] + agg_ref[1]) * dinv + b_ref[...], 0.0)
    y_ref[...] = jnp.dot(h, w_ref[...], preferred_element_type=jnp.float32) * dinv


def _mm3_body(agg_ref, dinv_ref, b_ref, wl_ref, bl_ref, out_ref):
    dinv = dinv_ref[...][:, 0:1]
    h = jnp.maximum((agg_ref[0] + agg_ref[1]) * dinv + b_ref[...], 0.0)
    logits = jnp.dot(h, wl_ref[...], preferred_element_type=jnp.float32) + bl_ref[...]
    col = lax.broadcasted_iota(jnp.int32, logits.shape, 1)
    valid = col < N_CLS
    masked = jnp.where(valid, logits, -jnp.inf)
    m = jnp.max(masked, axis=1, keepdims=True)
    e = jnp.where(valid, jnp.exp(logits - m), 0.0)
    lse = jnp.log(jnp.sum(e, axis=1, keepdims=True)) + m
    out_ref[...] = (logits - lse)[:, :N_CLS]


def kernel(x, edge_index, W1, b1, W2, b2, Wl, bl):
    edges = edge_index.reshape(2, NW, CHUNKS, K)

    ones16 = jnp.ones((RPT, 16), jnp.float32)
    zeros16 = jnp.zeros((RPT, 16), jnp.float32)
    zerosD = jnp.zeros((RPT, D_FEAT), jnp.float32)

    deg2 = _deg_pass(edges, ones16, zeros16)

    grid = (N_NODES // _R,)
    y1, dinv16 = pl.pallas_call(
        _mm1_body,
        grid=grid,
        in_specs=[
            pl.BlockSpec((_R, D_FEAT), lambda i: (i, 0)),
            pl.BlockSpec((D_FEAT, D_FEAT), lambda i: (0, 0)),
            pl.BlockSpec((NC, _R, 16), lambda i: (0, i, 0)),
        ],
        out_specs=[
            pl.BlockSpec((_R, D_FEAT), lambda i: (i, 0)),
            pl.BlockSpec((_R, 16), lambda i: (i, 0)),
        ],
        out_shape=[
            jax.ShapeDtypeStruct((N_NODES, D_FEAT), jnp.float32),
            jax.ShapeDtypeStruct((N_NODES, 16), jnp.float32),
        ],
    )(x, W1, deg2)

    agg1 = _edge_pass(y1, edges, zerosD)

    b1r = b1.reshape(1, D_FEAT)
    y2 = pl.pallas_call(
        _mm2_body,
        grid=grid,
        in_specs=[
            pl.BlockSpec((NC, _R, D_FEAT), lambda i: (0, i, 0)),
            pl.BlockSpec((_R, 16), lambda i: (i, 0)),
            pl.BlockSpec((1, D_FEAT), lambda i: (0, 0)),
            pl.BlockSpec((D_FEAT, D_FEAT), lambda i: (0, 0)),
        ],
        out_specs=pl.BlockSpec((_R, D_FEAT), lambda i: (i, 0)),
        out_shape=jax.ShapeDtypeStruct((N_NODES, D_FEAT), jnp.float32),
    )(agg1, dinv16, b1r, W2)

    agg2 = _edge_pass(y2, edges, zerosD)

    b2r = b2.reshape(1, D_FEAT)
    Wlp = jnp.zeros((D_FEAT, D_FEAT), jnp.float32).at[:, :N_CLS].set(Wl)
    blp = jnp.zeros((1, D_FEAT), jnp.float32).at[0, :N_CLS].set(bl)
    outp = pl.pallas_call(
        _mm3_body,
        grid=grid,
        in_specs=[
            pl.BlockSpec((NC, _R, D_FEAT), lambda i: (0, i, 0)),
            pl.BlockSpec((_R, 16), lambda i: (i, 0)),
            pl.BlockSpec((1, D_FEAT), lambda i: (0, 0)),
            pl.BlockSpec((D_FEAT, D_FEAT), lambda i: (0, 0)),
            pl.BlockSpec((1, D_FEAT), lambda i: (0, 0)),
        ],
        out_specs=pl.BlockSpec((_R, N_CLS), lambda i: (i, 0)),
        out_shape=jax.ShapeDtypeStruct((N_NODES, N_CLS), jnp.float32),
    )(agg2, dinv16, b2r, Wlp, blp)

    return outp
